# Initial kernel scaffold; baseline (speedup 1.0000x reference)
import jax
import jax.numpy as jnp
from jax.experimental import pallas as pl


def kernel(x_job, x_station, x_machine, x_robot, alpha, actions, params, edges):
    raise NotImplementedError("write your pallas kernel here")


# speedup vs baseline: 1.0194x; 1.0194x over previous
import functools

import jax
import jax.numpy as jnp
from jax.experimental import pallas as pl
from jax.experimental.pallas import tpu as pltpu

B = 1024; J = 16; NJ = B * J; NS = 3 * B; NM = 2 * B; NR = B
DJ = 128; DO = 64; H = 4; GD = 128; A = 4096


def _gat(xs, xd, ei, p, nd):
    od = p["b"].shape[0]; C = od // H
    hs = (xs @ p["Ws"]).reshape(-1, H, C)
    a_s = (hs * p["as"][None]).sum(-1)
    a_d = ((xd @ p["Wd"]).reshape(-1, H, C) * p["ad"][None]).sum(-1)
    src, dst = ei[0], ei[1]
    e = jax.nn.leaky_relu(a_s[src] + a_d[dst], 0.2)
    ee = jnp.exp(e)
    den = jax.ops.segment_sum(ee, dst, num_segments=nd)
    num = jax.ops.segment_sum(hs[src] * ee[:, :, None], dst, num_segments=nd)
    out = num / (den[:, :, None] + 1e-16)
    return out.reshape(nd, od) + p["b"]


def _ln(x, g, b):
    mu = x.mean(-1, keepdims=True)
    v = ((x - mu) ** 2).mean(-1, keepdims=True)
    return (x - mu) / jnp.sqrt(v + 1e-5) * g + b


def _mlp_body(feat_ref, w1_ref, b1_ref, w2_ref, b2_ref, w3_ref, b3_ref, o_ref):
    h = jnp.maximum(feat_ref[...] @ w1_ref[...] + b1_ref[...], 0.0)
    h = jnp.maximum(h @ w2_ref[...] + b2_ref[...], 0.0)
    o_ref[...] = h @ w3_ref[...] + b3_ref[...]


def _q_mlp(feat, P):
    F = feat.shape[1]
    w3 = jnp.pad(P["q3_W"], ((0, 0), (0, 127)))
    b3 = jnp.pad(P["q3_b"], (0, 127))
    out = pl.pallas_call(
        _mlp_body,
        out_shape=jax.ShapeDtypeStruct((A, 128), jnp.float32),
    )(feat, P["q1_W"], P["q1_b"], P["q2_W"], P["q2_b"], w3, b3)
    return out[:, 0]


def kernel(x_job, x_station, x_machine, x_robot, alpha, actions, params, edges):
    P = params
    relu = jax.nn.relu
    hj = relu(x_job @ P["lj_W"] + P["lj_b"])
    hs = relu(x_station @ P["ls_W"] + P["ls_b"])
    hm = relu(x_machine @ P["lm_W"] + P["lm_b"])
    hr = relu(x_robot @ P["lr_W"] + P["lr_b"])
    E = edges
    for jl, ol in (("j1", "o1"), ("j2", "o2")):
        pj = P[jl]
        msg = (_gat(hs, hj, E["cl"], pj["cl"], NJ) + _gat(hs, hj, E["ld"], pj["ld"], NJ)
               + _gat(hm, hj, E["we"], pj["we"], NJ) + _gat(hm, hj, E["ex"], pj["ex"], NJ)
               + _gat(hr, hj, E["hd"], pj["hd"], NJ))
        hj = _ln(relu(msg + hj), pj["ln_g"], pj["ln_b"])
        po = P[ol]
        ms = _gat(hj, hs, E["cbl"], po["cbl"], NS) + _gat(hj, hs, E["li"], po["li"], NS)
        mm = _gat(hj, hm, E["nd"], po["nd"], NM) + _gat(hj, hm, E["eb"], po["eb"], NM)
        mr = _gat(hj, hr, E["hb"], po["hb"], NR)
        hs = _ln(relu(ms + hs), po["ln_gs"], po["ln_bs"])
        hm = _ln(relu(mm + hm), po["ln_gm"], po["ln_bm"])
        hr = _ln(relu(mr + hr), po["ln_gr"], po["ln_br"])
    h_nodes = jnp.concatenate([hs.reshape(B, 3 * DO), hm.reshape(B, 2 * DO), hr.reshape(B, DO)], axis=1)
    gate = (hj @ P["gate_W"] + P["gate_b"])[:, 0].reshape(B, J)
    ge = jnp.exp(gate)
    w = ge / (ge.sum(-1, keepdims=True) + 1e-16)
    mean_jobs = (hj.reshape(B, J, DJ) * w[:, :, None]).sum(1)
    h_global = relu(jnp.concatenate([h_nodes, mean_jobs], axis=1) @ P["gl_W"] + P["gl_b"])
    job_ids = actions[:, 0]
    graph_ids = job_ids // J
    gji = job_ids + graph_ids * J
    emb = hj[gji]
    hg = h_global[graph_ids]
    aA = jnp.broadcast_to(alpha.reshape(1, 1).astype(jnp.float32), (A, 1))
    feat = jnp.concatenate([emb, hg, actions[:, 1:2].astype(jnp.float32),
                            actions[:, 2:3].astype(jnp.float32), aA], axis=1)
    return _q_mlp(feat, P)


# speedup vs baseline: 37.6256x; 36.9098x over previous
import functools

import jax
import jax.numpy as jnp
from jax import lax
from jax.experimental import pallas as pl
from jax.experimental.pallas import tpu as pltpu
from jax.experimental.pallas import tpu_sc as plsc

B = 1024; J = 16; NJ = B * J; NS = 3 * B; NM = 2 * B; NR = B
DJ = 128; DO = 64; H = 4; GD = 128; A = 4096

CH = 128
ZR = 64


def _build_phase(E_list, ns_list, acc_base, rows_g, od2):
    W = od2 + 16
    nrel = len(E_list)
    tab_base = [0]
    for ns in ns_list[:-1]:
        tab_base.append(tab_base[-1] + ns)
    T = tab_base[-1] + ns_list[-1]
    jstyle = all(b == 0 for b in acc_base)
    if jstyle:
        groups = [[r] for r in range(nrel)]
        ad_base = [rows_g * r for r in range(nrel)]
        ADR = rows_g * nrel
        out_base = [rows_g * r for r in range(nrel)]
        R_out = rows_g * nrel
    else:
        groups = [list(range(nrel))]
        ad_base = [0] * nrel
        ADR = rows_g
        out_base = [0]
        R_out = rows_g
    e_base = [0]
    for E in E_list[:-1]:
        e_base.append(e_base[-1] + E)
    MAXNS = max(ns_list)
    C2 = od2 // 2

    mesh = plsc.VectorSubcoreMesh(core_axis_name="c", subcore_axis_name="s")

    @functools.partial(
        pl.kernel, mesh=mesh,
        compiler_params=pltpu.CompilerParams(
            needs_layout_passes=False, use_tc_tiling_on_sc=False),
        out_type=jax.ShapeDtypeStruct((2 * R_out, 128), jnp.float32),
        scratch_types=[
            pltpu.VMEM((CH, 128), jnp.float32),
            pltpu.VMEM((CH, W), jnp.float32),
            pltpu.VMEM((CH,), jnp.int32),
            pltpu.VMEM((CH,), jnp.int32),
            pltpu.VMEM((CH,), jnp.int32),
            pltpu.VMEM((CH,), jnp.int32),
            pltpu.VMEM((CH,), jnp.float32),
            pltpu.VMEM((CH,), jnp.float32),
            pltpu.VMEM((ZR, W), jnp.float32),
            pltpu.VMEM((MAXNS * 2,), jnp.float32),
            pltpu.VMEM_SHARED((rows_g * 2,), jnp.float32),
            pltpu.VMEM_SHARED((rows_g, W), jnp.float32),
            pltpu.SemaphoreType.DMA,
            pltpu.SemaphoreType.DMA,
        ])
    def phase(src_hbm, dst_hbm, hs_hbm, as_hbm, ad_hbm, out_hbm,
              rows_v, staged, src_v, dst_v, i0b, i1b, ad0v, ad1v,
              zbuf, asb, adsh, acc, sem, sem_a):
        c = lax.axis_index("c")
        s = lax.axis_index("s")
        cT = c * T
        co = c * od2

        def zrow(i, _):
            for k in range(W // 16):
                zbuf[i, pl.ds(k * 16, 16)] = jnp.zeros((16,), jnp.float32)
            return ()
        lax.fori_loop(0, ZR, zrow, ())

        def zpad(i, _):
            staged[i, pl.ds(od2, 16)] = jnp.zeros((16,), jnp.float32)
            return ()
        lax.fori_loop(0, CH, zpad, ())

        for g, rels in enumerate(groups):
            rg = rows_g
            rslice = rg // 16
            r0 = s * rslice

            def zacc(t, _):
                pltpu.sync_copy(zbuf, acc.at[pl.ds(r0 + t * ZR, ZR)])
                return ()
            lax.fori_loop(0, rslice // ZR, zacc, ())
            adr0 = s * (rg * 2 // 16)
            pltpu.sync_copy(
                ad_hbm.at[pl.ds((c * ADR + ad_base[g]) * 2 + adr0,
                                rg * 2 // 16)],
                adsh.at[pl.ds(adr0, rg * 2 // 16)])
            plsc.subcore_barrier()

            for r in rels:
                E = E_list[r]; ns = ns_list[r]; tb = tab_base[r]
                ept = E // 16
                eb = e_base[r] + s * ept
                pltpu.sync_copy(as_hbm.at[pl.ds((cT + tb) * 2, ns * 2)],
                                asb.at[pl.ds(0, ns * 2)])

                def chunk(ci, _):
                    cb = eb + ci * CH
                    pltpu.sync_copy(src_hbm.at[pl.ds(cb, CH)], src_v)
                    pltpu.sync_copy(dst_hbm.at[pl.ds(cb, CH)], dst_v)

                    gat = pltpu.async_copy(hs_hbm.at[src_v], rows_v, sem)

                    def mkidx(j, _):
                        d16 = dst_v[pl.ds(j * 16, 16)]
                        i0b[pl.ds(j * 16, 16)] = d16 * 2
                        i1b[pl.ds(j * 16, 16)] = d16 * 2 + 1
                        return ()
                    lax.fori_loop(0, CH // 16, mkidx, ())
                    pltpu.async_copy(adsh.at[i0b], ad0v, sem_a).wait()
                    pltpu.async_copy(adsh.at[i1b], ad1v, sem_a).wait()
                    gat.wait()

                    zero16 = jnp.zeros((16,), jnp.int32)

                    def att(gi, _):
                        s16 = src_v[pl.ds(gi * 16, 16)] - tb
                        as0 = plsc.load_gather(asb, [s16 * 2])
                        as1 = plsc.load_gather(asb, [s16 * 2 + 1])
                        ad0 = ad0v[pl.ds(gi * 16, 16)]
                        ad1 = ad1v[pl.ds(gi * 16, 16)]
                        e0 = as0 + ad0
                        e1 = as1 + ad1
                        ee0 = jnp.exp(jnp.maximum(e0, e0 * 0.2))
                        ee1 = jnp.exp(jnp.maximum(e1, e1 * 0.2))
                        evec = gi * 16 + lax.iota(jnp.int32, 16)
                        plsc.store_scatter(
                            staged, [evec, zero16 + od2], ee0)
                        plsc.store_scatter(
                            staged, [evec, zero16 + (od2 + 1)], ee1)
                        return ()
                    lax.fori_loop(0, CH // 16, att, ())

                    def scale(e, _):
                        eev = staged[e, pl.ds(od2, 16)]
                        ee0 = eev[0]
                        ee1 = eev[1]
                        for k in range(od2 // 16):
                            v = rows_v[e, pl.ds(co + k * 16, 16)]
                            f = ee0 if k < C2 // 16 else ee1
                            staged[e, pl.ds(k * 16, 16)] = v * f
                        return ()
                    lax.fori_loop(0, CH, scale, ())

                    pltpu.sync_copy(staged, acc.at[dst_v], add=True)
                    return ()
                lax.fori_loop(0, ept // CH, chunk, ())

            plsc.subcore_barrier()
            ob = c * R_out + out_base[g]

            def flush(t, _):
                pltpu.sync_copy(acc.at[pl.ds(r0 + t * ZR, ZR)],
                                out_hbm.at[pl.ds(ob + r0 + t * ZR, ZR),
                                           pl.ds(0, W)])
                return ()
            lax.fori_loop(0, rslice // ZR, flush, ())
            plsc.subcore_barrier()

    return phase


_J_E = [65536, 65536, 65536, 65536, 32768]
_J_NS = [NS, NS, NM, NM, NR]
_O_E = [65536, 65536, 65536, 65536, 32768]
_O_NS = [NJ] * 5
_O_BASE = [0, NS, 2 * NS, 2 * NS + NM, 2 * NS + 2 * NM]
_O_ROWS = 2 * NS + 2 * NM + NR

_phase_j = _build_phase(_J_E, _J_NS, [0] * 5, NJ, DJ // 2)
_phase_o = _build_phase(_O_E, _O_NS, _O_BASE, _O_ROWS, DO // 2)


def _fold_a(Wmat, avec, od):
    C = od // H
    return jnp.einsum("shc,hc->sh", Wmat.reshape(-1, H, C), avec)


def _prep_tables(xs_list, xd, ps, od):
    hs_l, as_l, ad_l = [], [], []
    for xs, p in zip(xs_list, ps):
        h = xs @ p["Ws"]
        if od < 128:
            h = jnp.pad(h, ((0, 0), (0, 128 - od)))
        hs_l.append(h)
        as_l.append(xs @ _fold_a(p["Ws"], p["as"], od))
        ad_l.append(xd @ _fold_a(p["Wd"], p["ad"], od))
    hs_tab = jnp.concatenate(hs_l)
    as_tab = jnp.concatenate(
        [a[:, 2 * c:2 * c + 2] for c in (0, 1) for a in as_l])
    ad_tab = jnp.concatenate(
        [a[:, 2 * c:2 * c + 2] for c in (0, 1) for a in ad_l])
    return hs_tab, as_tab, ad_tab


def _prep_tables_o(xs, xd_list, ps, od):
    hs_l, as_l, ad_l = [], [], []
    for xd, p in zip(xd_list, ps):
        h = xs @ p["Ws"]
        if od < 128:
            h = jnp.pad(h, ((0, 0), (0, 128 - od)))
        hs_l.append(h)
        as_l.append(xs @ _fold_a(p["Ws"], p["as"], od))
        ad_l.append(xd @ _fold_a(p["Wd"], p["ad"], od))
    hs_tab = jnp.concatenate(hs_l)
    as_tab = jnp.concatenate(
        [a[:, 2 * c:2 * c + 2] for c in (0, 1) for a in as_l])
    ad_tab = jnp.concatenate(
        [a[:, 2 * c:2 * c + 2] for c in (0, 1) for a in ad_l])
    return hs_tab, as_tab, ad_tab


def _combine_j(raw, ps):
    od2 = DJ // 2
    r = raw.reshape(2, 5, NJ, 128)
    num = r[:, :, :, :od2].reshape(2, 5, NJ, 2, od2 // 2)
    den = r[:, :, :, od2:od2 + 2].reshape(2, 5, NJ, 2, 1)
    out = num / (den + 1e-16)
    out = out.reshape(2, 5, NJ, od2).transpose(1, 2, 0, 3).reshape(5, NJ, DJ)
    bsum = sum(p["b"] for p in ps)
    return out.sum(0) + bsum


def _combine_o(raw, ps):
    od2 = DO // 2
    r = raw.reshape(2, _O_ROWS, 128)
    num = r[:, :, :od2].reshape(2, _O_ROWS, 2, od2 // 2)
    den = r[:, :, od2:od2 + 2].reshape(2, _O_ROWS, 2, 1)
    out = num / (den + 1e-16)
    out = out.reshape(2, _O_ROWS, od2).transpose(1, 0, 2).reshape(_O_ROWS, DO)
    parts = []
    o = 0
    for p, n in zip(ps, [NS, NS, NM, NM, NR]):
        parts.append(out[o:o + n] + p["b"])
        o += n
    return parts


def _ln(x, g, b):
    mu = x.mean(-1, keepdims=True)
    v = ((x - mu) ** 2).mean(-1, keepdims=True)
    return (x - mu) / jnp.sqrt(v + 1e-5) * g + b


def _mlp_body(feat_ref, w1_ref, b1_ref, w2_ref, b2_ref, w3_ref, b3_ref, o_ref):
    h = jnp.maximum(feat_ref[...] @ w1_ref[...] + b1_ref[...], 0.0)
    h = jnp.maximum(h @ w2_ref[...] + b2_ref[...], 0.0)
    o_ref[...] = h @ w3_ref[...] + b3_ref[...]


def _q_mlp(feat, P):
    w3 = jnp.pad(P["q3_W"], ((0, 0), (0, 127)))
    b3 = jnp.pad(P["q3_b"], (0, 127))
    out = pl.pallas_call(
        _mlp_body,
        out_shape=jax.ShapeDtypeStruct((A, 128), jnp.float32),
    )(feat, P["q1_W"], P["q1_b"], P["q2_W"], P["q2_b"], w3, b3)
    return out[:, 0]


_J_RELS = ("cl", "ld", "we", "ex", "hd")
_O_RELS = ("cbl", "li", "nd", "eb", "hb")


def kernel(x_job, x_station, x_machine, x_robot, alpha, actions, params, edges):
    P = params
    relu = jax.nn.relu
    hj = relu(x_job @ P["lj_W"] + P["lj_b"])
    hs = relu(x_station @ P["ls_W"] + P["ls_b"])
    hm = relu(x_machine @ P["lm_W"] + P["lm_b"])
    hr = relu(x_robot @ P["lr_W"] + P["lr_b"])

    jt = [0, NS, NS, 0, 0]
    j_tb = [0, NS, 2 * NS, 2 * NS + NM, 2 * NS + 2 * NM]
    src_j = jnp.concatenate(
        [edges[r][0].astype(jnp.int32) + j_tb[i] for i, r in enumerate(_J_RELS)])
    dst_j = jnp.concatenate(
        [edges[r][1].astype(jnp.int32) for r in _J_RELS])
    o_tb = [NJ * i for i in range(5)]
    src_o = jnp.concatenate(
        [edges[r][0].astype(jnp.int32) + o_tb[i] for i, r in enumerate(_O_RELS)])
    dst_o = jnp.concatenate(
        [edges[r][1].astype(jnp.int32) + _O_BASE[i] for i, r in enumerate(_O_RELS)])

    for jl, ol in (("j1", "o1"), ("j2", "o2")):
        pj = P[jl]
        ps_j = [pj[r] for r in _J_RELS]
        hs_tab, as_tab, ad_tab = _prep_tables([hs, hs, hm, hm, hr], hj, ps_j, DJ)
        raw = _phase_j(src_j, dst_j, hs_tab,
                       as_tab.reshape(-1), ad_tab.reshape(-1))
        msg = _combine_j(raw, ps_j)
        hj = _ln(relu(msg + hj), pj["ln_g"], pj["ln_b"])

        po = P[ol]
        ps_o = [po[r] for r in _O_RELS]
        hs_tab, as_tab, ad_tab = _prep_tables_o(hj, [hs, hs, hm, hm, hr], ps_o, DO)
        raw = _phase_o(src_o, dst_o, hs_tab,
                       as_tab.reshape(-1), ad_tab.reshape(-1))
        mss, mli, mnd, meb, mhb = _combine_o(raw, ps_o)
        ms = mss + mli
        mm = mnd + meb
        mr = mhb
        hs = _ln(relu(ms + hs), po["ln_gs"], po["ln_bs"])
        hm = _ln(relu(mm + hm), po["ln_gm"], po["ln_bm"])
        hr = _ln(relu(mr + hr), po["ln_gr"], po["ln_br"])

    h_nodes = jnp.concatenate(
        [hs.reshape(B, 3 * DO), hm.reshape(B, 2 * DO), hr.reshape(B, DO)], axis=1)
    gate = (hj @ P["gate_W"] + P["gate_b"])[:, 0].reshape(B, J)
    ge = jnp.exp(gate)
    w = ge / (ge.sum(-1, keepdims=True) + 1e-16)
    mean_jobs = (hj.reshape(B, J, DJ) * w[:, :, None]).sum(1)
    h_global = relu(jnp.concatenate([h_nodes, mean_jobs], axis=1) @ P["gl_W"] + P["gl_b"])
    job_ids = actions[:, 0]
    graph_ids = job_ids // J
    gji = job_ids + graph_ids * J
    emb = hj[gji]
    hg = h_global[graph_ids]
    aA = jnp.broadcast_to(alpha.reshape(1, 1).astype(jnp.float32), (A, 1))
    feat = jnp.concatenate([emb, hg, actions[:, 1:2].astype(jnp.float32),
                            actions[:, 2:3].astype(jnp.float32), aA], axis=1)
    return _q_mlp(feat, P)


# speedup vs baseline: 40.1581x; 1.0673x over previous
import functools

import jax
import jax.numpy as jnp
from jax import lax
from jax.experimental import pallas as pl
from jax.experimental.pallas import tpu as pltpu
from jax.experimental.pallas import tpu_sc as plsc

B = 1024; J = 16; NJ = B * J; NS = 3 * B; NM = 2 * B; NR = B
DJ = 128; DO = 64; H = 4; GD = 128; A = 4096

CH = 64
ZR = 64


def _build_phase(E_list, ns_list, acc_base, rows_g, od2):
    W = od2 + 16
    nrel = len(E_list)
    tab_base = [0]
    for ns in ns_list[:-1]:
        tab_base.append(tab_base[-1] + ns)
    T = tab_base[-1] + ns_list[-1]
    jstyle = all(b == 0 for b in acc_base)
    if jstyle:
        groups = [[r] for r in range(nrel)]
        ad_base = [rows_g * r for r in range(nrel)]
        ADR = rows_g * nrel
        out_base = [rows_g * r for r in range(nrel)]
        R_out = rows_g * nrel
    else:
        groups = [list(range(nrel))]
        ad_base = [0] * nrel
        ADR = rows_g
        out_base = [0]
        R_out = rows_g
    e_base = [0]
    for E in E_list[:-1]:
        e_base.append(e_base[-1] + E)
    MAXNS = max(ns_list)
    C2 = od2 // 2

    mesh = plsc.VectorSubcoreMesh(core_axis_name="c", subcore_axis_name="s")

    @functools.partial(
        pl.kernel, mesh=mesh,
        compiler_params=pltpu.CompilerParams(
            needs_layout_passes=False, use_tc_tiling_on_sc=False),
        out_type=jax.ShapeDtypeStruct((2 * R_out, 128), jnp.float32),
        scratch_types=[
            pltpu.VMEM((CH, 128), jnp.float32),
            pltpu.VMEM((CH, 128), jnp.float32),
            pltpu.VMEM((CH, W), jnp.float32),
            pltpu.VMEM((CH,), jnp.int32),
            pltpu.VMEM((CH,), jnp.int32),
            pltpu.VMEM((CH,), jnp.int32),
            pltpu.VMEM((CH,), jnp.int32),
            pltpu.VMEM((CH,), jnp.int32),
            pltpu.VMEM((CH,), jnp.int32),
            pltpu.VMEM((CH,), jnp.int32),
            pltpu.VMEM((CH,), jnp.int32),
            pltpu.VMEM((CH,), jnp.float32),
            pltpu.VMEM((CH,), jnp.float32),
            pltpu.VMEM((CH,), jnp.float32),
            pltpu.VMEM((CH,), jnp.float32),
            pltpu.VMEM((ZR, W), jnp.float32),
            pltpu.VMEM((MAXNS * 2,), jnp.float32),
            pltpu.VMEM_SHARED((rows_g * 2,), jnp.float32),
            pltpu.VMEM_SHARED((rows_g, W), jnp.float32),
            pltpu.SemaphoreType.DMA,
            pltpu.SemaphoreType.DMA,
            pltpu.SemaphoreType.DMA,
            pltpu.SemaphoreType.DMA,
        ])
    def phase(src_hbm, dst_hbm, hs_hbm, as_hbm, ad_hbm, out_hbm,
              rows_v0, rows_v1, staged, src_v0, src_v1, dst_v0, dst_v1,
              i0b0, i0b1, i1b0, i1b1, ad0v0, ad0v1, ad1v0, ad1v1,
              zbuf, asb, adsh, acc, semr0, semr1, sema0, sema1):
        c = lax.axis_index("c")
        s = lax.axis_index("s")
        cT = c * T
        co = c * od2

        def zrow(i, _):
            for k in range(W // 16):
                zbuf[i, pl.ds(k * 16, 16)] = jnp.zeros((16,), jnp.float32)
            return ()
        lax.fori_loop(0, ZR, zrow, ())

        def zpad(i, _):
            staged[i, pl.ds(od2, 16)] = jnp.zeros((16,), jnp.float32)
            return ()
        lax.fori_loop(0, CH, zpad, ())

        for g, rels in enumerate(groups):
            rg = rows_g
            rslice = rg // 16
            r0 = s * rslice

            def zacc(t, _):
                pltpu.sync_copy(zbuf, acc.at[pl.ds(r0 + t * ZR, ZR)])
                return ()
            lax.fori_loop(0, rslice // ZR, zacc, ())
            adr0 = s * (rg * 2 // 16)
            pltpu.sync_copy(
                ad_hbm.at[pl.ds((c * ADR + ad_base[g]) * 2 + adr0,
                                rg * 2 // 16)],
                adsh.at[pl.ds(adr0, rg * 2 // 16)])
            plsc.subcore_barrier()

            for r in rels:
                E = E_list[r]; ns = ns_list[r]; tb = tab_base[r]
                ept = E // 16
                eb = e_base[r] + s * ept
                nch = ept // CH
                npair = nch // 2
                pltpu.sync_copy(as_hbm.at[pl.ds((cT + tb) * 2, ns * 2)],
                                asb.at[pl.ds(0, ns * 2)])
                bufs = (
                    (rows_v0, src_v0, dst_v0, i0b0, i1b0, ad0v0, ad1v0,
                     semr0, sema0),
                    (rows_v1, src_v1, dst_v1, i0b1, i1b1, ad0v1, ad1v1,
                     semr1, sema1),
                )

                def issue(ci, bf):
                    rows_v, src_v, dst_v, i0b, i1b, ad0v, ad1v, smr, sma = bf
                    cb = eb + ci * CH
                    pltpu.sync_copy(src_hbm.at[pl.ds(cb, CH)], src_v)
                    pltpu.sync_copy(dst_hbm.at[pl.ds(cb, CH)], dst_v)

                    def mkidx(j, _):
                        d16 = dst_v[pl.ds(j * 16, 16)]
                        i0b[pl.ds(j * 16, 16)] = d16 * 2
                        i1b[pl.ds(j * 16, 16)] = d16 * 2 + 1
                        return ()
                    lax.fori_loop(0, CH // 16, mkidx, ())
                    pltpu.async_copy(hs_hbm.at[src_v], rows_v, smr)
                    pltpu.async_copy(adsh.at[i0b], ad0v, sma)
                    pltpu.async_copy(adsh.at[i1b], ad1v, sma)

                def consume(bf):
                    rows_v, src_v, dst_v, i0b, i1b, ad0v, ad1v, smr, sma = bf
                    pltpu.make_async_copy(hs_hbm.at[src_v], rows_v, smr).wait()
                    pltpu.make_async_copy(adsh.at[i0b], ad0v, sma).wait()
                    pltpu.make_async_copy(adsh.at[i1b], ad1v, sma).wait()

                    zero16 = jnp.zeros((16,), jnp.int32)

                    def att(gi, _):
                        s16 = src_v[pl.ds(gi * 16, 16)] - tb
                        as0 = plsc.load_gather(asb, [s16 * 2])
                        as1 = plsc.load_gather(asb, [s16 * 2 + 1])
                        ad0 = ad0v[pl.ds(gi * 16, 16)]
                        ad1 = ad1v[pl.ds(gi * 16, 16)]
                        e0 = as0 + ad0
                        e1 = as1 + ad1
                        ee0 = jnp.exp(jnp.maximum(e0, e0 * 0.2))
                        ee1 = jnp.exp(jnp.maximum(e1, e1 * 0.2))
                        evec = gi * 16 + lax.iota(jnp.int32, 16)
                        plsc.store_scatter(
                            staged, [evec, zero16 + od2], ee0)
                        plsc.store_scatter(
                            staged, [evec, zero16 + (od2 + 1)], ee1)
                        return ()
                    lax.fori_loop(0, CH // 16, att, ())

                    def scale(e, _):
                        eev = staged[e, pl.ds(od2, 16)]
                        ee0 = eev[0]
                        ee1 = eev[1]
                        for k in range(od2 // 16):
                            v = rows_v[e, pl.ds(co + k * 16, 16)]
                            f = ee0 if k < C2 // 16 else ee1
                            staged[e, pl.ds(k * 16, 16)] = v * f
                        return ()
                    lax.fori_loop(0, CH, scale, ())

                    pltpu.sync_copy(staged, acc.at[dst_v], add=True)

                issue(0, bufs[0])

                def pair(i, _):
                    issue(i * 2 + 1, bufs[1])
                    consume(bufs[0])

                    @pl.when(i < npair - 1)
                    def _():
                        issue(i * 2 + 2, bufs[0])
                    consume(bufs[1])
                    return ()
                lax.fori_loop(0, npair, pair, ())

            plsc.subcore_barrier()
            ob = c * R_out + out_base[g]

            def flush(t, _):
                pltpu.sync_copy(acc.at[pl.ds(r0 + t * ZR, ZR)],
                                out_hbm.at[pl.ds(ob + r0 + t * ZR, ZR),
                                           pl.ds(0, W)])
                return ()
            lax.fori_loop(0, rslice // ZR, flush, ())
            plsc.subcore_barrier()

    return phase


_J_E = [65536, 65536, 65536, 65536, 32768]
_J_NS = [NS, NS, NM, NM, NR]
_O_E = [65536, 65536, 65536, 65536, 32768]
_O_NS = [NJ] * 5
_O_BASE = [0, NS, 2 * NS, 2 * NS + NM, 2 * NS + 2 * NM]
_O_ROWS = 2 * NS + 2 * NM + NR

_phase_j = _build_phase(_J_E, _J_NS, [0] * 5, NJ, DJ // 2)
_phase_o = _build_phase(_O_E, _O_NS, _O_BASE, _O_ROWS, DO // 2)


def _fold_a(Wmat, avec, od):
    C = od // H
    return jnp.einsum("shc,hc->sh", Wmat.reshape(-1, H, C), avec)


def _prep_tables(xs_list, xd, ps, od):
    hs_l, as_l, ad_l = [], [], []
    for xs, p in zip(xs_list, ps):
        h = xs @ p["Ws"]
        if od < 128:
            h = jnp.pad(h, ((0, 0), (0, 128 - od)))
        hs_l.append(h)
        as_l.append(xs @ _fold_a(p["Ws"], p["as"], od))
        ad_l.append(xd @ _fold_a(p["Wd"], p["ad"], od))
    hs_tab = jnp.concatenate(hs_l)
    as_tab = jnp.concatenate(
        [a[:, 2 * c:2 * c + 2] for c in (0, 1) for a in as_l])
    ad_tab = jnp.concatenate(
        [a[:, 2 * c:2 * c + 2] for c in (0, 1) for a in ad_l])
    return hs_tab, as_tab, ad_tab


def _prep_tables_o(xs, xd_list, ps, od):
    hs_l, as_l, ad_l = [], [], []
    for xd, p in zip(xd_list, ps):
        h = xs @ p["Ws"]
        if od < 128:
            h = jnp.pad(h, ((0, 0), (0, 128 - od)))
        hs_l.append(h)
        as_l.append(xs @ _fold_a(p["Ws"], p["as"], od))
        ad_l.append(xd @ _fold_a(p["Wd"], p["ad"], od))
    hs_tab = jnp.concatenate(hs_l)
    as_tab = jnp.concatenate(
        [a[:, 2 * c:2 * c + 2] for c in (0, 1) for a in as_l])
    ad_tab = jnp.concatenate(
        [a[:, 2 * c:2 * c + 2] for c in (0, 1) for a in ad_l])
    return hs_tab, as_tab, ad_tab


def _combine_j(raw, ps):
    od2 = DJ // 2
    r = raw.reshape(2, 5, NJ, 128)
    num = r[:, :, :, :od2].reshape(2, 5, NJ, 2, od2 // 2)
    den = r[:, :, :, od2:od2 + 2].reshape(2, 5, NJ, 2, 1)
    out = num / (den + 1e-16)
    out = out.reshape(2, 5, NJ, od2).transpose(1, 2, 0, 3).reshape(5, NJ, DJ)
    bsum = sum(p["b"] for p in ps)
    return out.sum(0) + bsum


def _combine_o(raw, ps):
    od2 = DO // 2
    r = raw.reshape(2, _O_ROWS, 128)
    num = r[:, :, :od2].reshape(2, _O_ROWS, 2, od2 // 2)
    den = r[:, :, od2:od2 + 2].reshape(2, _O_ROWS, 2, 1)
    out = num / (den + 1e-16)
    out = out.reshape(2, _O_ROWS, od2).transpose(1, 0, 2).reshape(_O_ROWS, DO)
    parts = []
    o = 0
    for p, n in zip(ps, [NS, NS, NM, NM, NR]):
        parts.append(out[o:o + n] + p["b"])
        o += n
    return parts


def _ln(x, g, b):
    mu = x.mean(-1, keepdims=True)
    v = ((x - mu) ** 2).mean(-1, keepdims=True)
    return (x - mu) / jnp.sqrt(v + 1e-5) * g + b


def _mlp_body(feat_ref, w1_ref, b1_ref, w2_ref, b2_ref, w3_ref, b3_ref, o_ref):
    h = jnp.maximum(feat_ref[...] @ w1_ref[...] + b1_ref[...], 0.0)
    h = jnp.maximum(h @ w2_ref[...] + b2_ref[...], 0.0)
    o_ref[...] = h @ w3_ref[...] + b3_ref[...]


def _q_mlp(feat, P):
    w3 = jnp.pad(P["q3_W"], ((0, 0), (0, 127)))
    b3 = jnp.pad(P["q3_b"], (0, 127))
    out = pl.pallas_call(
        _mlp_body,
        out_shape=jax.ShapeDtypeStruct((A, 128), jnp.float32),
    )(feat, P["q1_W"], P["q1_b"], P["q2_W"], P["q2_b"], w3, b3)
    return out[:, 0]


_J_RELS = ("cl", "ld", "we", "ex", "hd")
_O_RELS = ("cbl", "li", "nd", "eb", "hb")


def kernel(x_job, x_station, x_machine, x_robot, alpha, actions, params, edges):
    P = params
    relu = jax.nn.relu
    hj = relu(x_job @ P["lj_W"] + P["lj_b"])
    hs = relu(x_station @ P["ls_W"] + P["ls_b"])
    hm = relu(x_machine @ P["lm_W"] + P["lm_b"])
    hr = relu(x_robot @ P["lr_W"] + P["lr_b"])

    jt = [0, NS, NS, 0, 0]
    j_tb = [0, NS, 2 * NS, 2 * NS + NM, 2 * NS + 2 * NM]
    src_j = jnp.concatenate(
        [edges[r][0].astype(jnp.int32) + j_tb[i] for i, r in enumerate(_J_RELS)])
    dst_j = jnp.concatenate(
        [edges[r][1].astype(jnp.int32) for r in _J_RELS])
    o_tb = [NJ * i for i in range(5)]
    src_o = jnp.concatenate(
        [edges[r][0].astype(jnp.int32) + o_tb[i] for i, r in enumerate(_O_RELS)])
    dst_o = jnp.concatenate(
        [edges[r][1].astype(jnp.int32) + _O_BASE[i] for i, r in enumerate(_O_RELS)])

    for jl, ol in (("j1", "o1"), ("j2", "o2")):
        pj = P[jl]
        ps_j = [pj[r] for r in _J_RELS]
        hs_tab, as_tab, ad_tab = _prep_tables([hs, hs, hm, hm, hr], hj, ps_j, DJ)
        raw = _phase_j(src_j, dst_j, hs_tab,
                       as_tab.reshape(-1), ad_tab.reshape(-1))
        msg = _combine_j(raw, ps_j)
        hj = _ln(relu(msg + hj), pj["ln_g"], pj["ln_b"])

        po = P[ol]
        ps_o = [po[r] for r in _O_RELS]
        hs_tab, as_tab, ad_tab = _prep_tables_o(hj, [hs, hs, hm, hm, hr], ps_o, DO)
        raw = _phase_o(src_o, dst_o, hs_tab,
                       as_tab.reshape(-1), ad_tab.reshape(-1))
        mss, mli, mnd, meb, mhb = _combine_o(raw, ps_o)
        ms = mss + mli
        mm = mnd + meb
        mr = mhb
        hs = _ln(relu(ms + hs), po["ln_gs"], po["ln_bs"])
        hm = _ln(relu(mm + hm), po["ln_gm"], po["ln_bm"])
        hr = _ln(relu(mr + hr), po["ln_gr"], po["ln_br"])

    h_nodes = jnp.concatenate(
        [hs.reshape(B, 3 * DO), hm.reshape(B, 2 * DO), hr.reshape(B, DO)], axis=1)
    gate = (hj @ P["gate_W"] + P["gate_b"])[:, 0].reshape(B, J)
    ge = jnp.exp(gate)
    w = ge / (ge.sum(-1, keepdims=True) + 1e-16)
    mean_jobs = (hj.reshape(B, J, DJ) * w[:, :, None]).sum(1)
    h_global = relu(jnp.concatenate([h_nodes, mean_jobs], axis=1) @ P["gl_W"] + P["gl_b"])
    job_ids = actions[:, 0]
    graph_ids = job_ids // J
    gji = job_ids + graph_ids * J
    emb = hj[gji]
    hg = h_global[graph_ids]
    aA = jnp.broadcast_to(alpha.reshape(1, 1).astype(jnp.float32), (A, 1))
    feat = jnp.concatenate([emb, hg, actions[:, 1:2].astype(jnp.float32),
                            actions[:, 2:3].astype(jnp.float32), aA], axis=1)
    return _q_mlp(feat, P)


# speedup vs baseline: 41.3957x; 1.0308x over previous
import functools

import jax
import jax.numpy as jnp
from jax import lax
from jax.experimental import pallas as pl
from jax.experimental.pallas import tpu as pltpu
from jax.experimental.pallas import tpu_sc as plsc

B = 1024; J = 16; NJ = B * J; NS = 3 * B; NM = 2 * B; NR = B
DJ = 128; DO = 64; H = 4; GD = 128; A = 4096

CH = 64
ZR = 64


def _build_phase(E_list, ns_list, acc_base, rows_g, od2):
    W = od2 + 16
    nrel = len(E_list)
    tab_base = [0]
    for ns in ns_list[:-1]:
        tab_base.append(tab_base[-1] + ns)
    T = tab_base[-1] + ns_list[-1]
    jstyle = all(b == 0 for b in acc_base)
    if jstyle:
        groups = [[r] for r in range(nrel)]
        ad_base = [rows_g * r for r in range(nrel)]
        ADR = rows_g * nrel
        out_base = [rows_g * r for r in range(nrel)]
        R_out = rows_g * nrel
    else:
        groups = [list(range(nrel))]
        ad_base = [0] * nrel
        ADR = rows_g
        out_base = [0]
        R_out = rows_g
    e_base = [0]
    for E in E_list[:-1]:
        e_base.append(e_base[-1] + E)
    MAXNS = max(ns_list)
    C2 = od2 // 2

    mesh = plsc.VectorSubcoreMesh(core_axis_name="c", subcore_axis_name="s")

    @functools.partial(
        pl.kernel, mesh=mesh,
        compiler_params=pltpu.CompilerParams(
            needs_layout_passes=False, use_tc_tiling_on_sc=False),
        out_type=jax.ShapeDtypeStruct((2 * R_out, 128), jnp.float32),
        scratch_types=[
            pltpu.VMEM((CH, 128), jnp.float32),
            pltpu.VMEM((CH, 128), jnp.float32),
            pltpu.VMEM((CH, W), jnp.float32),
            pltpu.VMEM((CH,), jnp.int32),
            pltpu.VMEM((CH,), jnp.int32),
            pltpu.VMEM((CH,), jnp.int32),
            pltpu.VMEM((CH,), jnp.int32),
            pltpu.VMEM((CH,), jnp.int32),
            pltpu.VMEM((CH,), jnp.int32),
            pltpu.VMEM((CH,), jnp.int32),
            pltpu.VMEM((CH,), jnp.int32),
            pltpu.VMEM((CH,), jnp.float32),
            pltpu.VMEM((CH,), jnp.float32),
            pltpu.VMEM((CH,), jnp.float32),
            pltpu.VMEM((CH,), jnp.float32),
            pltpu.VMEM((ZR, W), jnp.float32),
            pltpu.VMEM((MAXNS * 2,), jnp.float32),
            pltpu.VMEM_SHARED((rows_g * 2,), jnp.float32),
            pltpu.VMEM_SHARED((rows_g, W), jnp.float32),
            pltpu.SemaphoreType.DMA,
            pltpu.SemaphoreType.DMA,
            pltpu.SemaphoreType.DMA,
            pltpu.SemaphoreType.DMA,
        ])
    def phase(src_hbm, dst_hbm, hs_hbm, as_hbm, ad_hbm, out_hbm,
              rows_v0, rows_v1, staged, src_v0, src_v1, dst_v0, dst_v1,
              i0b0, i0b1, i1b0, i1b1, ad0v0, ad0v1, ad1v0, ad1v1,
              zbuf, asb, adsh, acc, semr0, semr1, sema0, sema1):
        c = lax.axis_index("c")
        s = lax.axis_index("s")
        cT = c * T
        co = c * od2

        def zrow(i, _):
            for k in range(W // 16):
                zbuf[i, pl.ds(k * 16, 16)] = jnp.zeros((16,), jnp.float32)
            return ()
        lax.fori_loop(0, ZR, zrow, ())

        def zpad(i, _):
            staged[i, pl.ds(od2, 16)] = jnp.zeros((16,), jnp.float32)
            return ()
        lax.fori_loop(0, CH, zpad, ())

        for g, rels in enumerate(groups):
            rg = rows_g
            rslice = rg // 16
            r0 = s * rslice

            def zacc(t, _):
                pltpu.sync_copy(zbuf, acc.at[pl.ds(r0 + t * ZR, ZR)])
                return ()
            lax.fori_loop(0, rslice // ZR, zacc, ())
            adr0 = s * (rg * 2 // 16)
            pltpu.sync_copy(
                ad_hbm.at[pl.ds((c * ADR + ad_base[g]) * 2 + adr0,
                                rg * 2 // 16)],
                adsh.at[pl.ds(adr0, rg * 2 // 16)])
            plsc.subcore_barrier()

            for r in rels:
                E = E_list[r]; ns = ns_list[r]; tb = tab_base[r]
                ept = E // 16
                eb = e_base[r] + s * ept
                nch = ept // CH
                npair = nch // 2
                pltpu.sync_copy(as_hbm.at[pl.ds((cT + tb) * 2, ns * 2)],
                                asb.at[pl.ds(0, ns * 2)])
                bufs = (
                    (rows_v0, src_v0, dst_v0, i0b0, i1b0, ad0v0, ad1v0,
                     semr0, sema0),
                    (rows_v1, src_v1, dst_v1, i0b1, i1b1, ad0v1, ad1v1,
                     semr1, sema1),
                )

                def issue(ci, bf):
                    rows_v, src_v, dst_v, i0b, i1b, ad0v, ad1v, smr, sma = bf
                    cb = eb + ci * CH
                    pltpu.sync_copy(src_hbm.at[pl.ds(cb, CH)], src_v)
                    pltpu.sync_copy(dst_hbm.at[pl.ds(cb, CH)], dst_v)

                    def mkidx(j, _):
                        d16 = dst_v[pl.ds(j * 16, 16)]
                        i0b[pl.ds(j * 16, 16)] = d16 * 2
                        i1b[pl.ds(j * 16, 16)] = d16 * 2 + 1
                        return ()
                    lax.fori_loop(0, CH // 16, mkidx, ())
                    pltpu.async_copy(hs_hbm.at[src_v], rows_v, smr)
                    pltpu.async_copy(adsh.at[i0b], ad0v, sma)
                    pltpu.async_copy(adsh.at[i1b], ad1v, sma)

                def consume(bf):
                    rows_v, src_v, dst_v, i0b, i1b, ad0v, ad1v, smr, sma = bf
                    pltpu.make_async_copy(hs_hbm.at[src_v], rows_v, smr).wait()
                    pltpu.make_async_copy(adsh.at[i0b], ad0v, sma).wait()
                    pltpu.make_async_copy(adsh.at[i1b], ad1v, sma).wait()

                    zero16 = jnp.zeros((16,), jnp.int32)

                    def att(gi, _):
                        s16 = src_v[pl.ds(gi * 16, 16)] - tb
                        as0 = plsc.load_gather(asb, [s16 * 2])
                        as1 = plsc.load_gather(asb, [s16 * 2 + 1])
                        ad0 = ad0v[pl.ds(gi * 16, 16)]
                        ad1 = ad1v[pl.ds(gi * 16, 16)]
                        e0 = as0 + ad0
                        e1 = as1 + ad1
                        ee0 = jnp.exp(jnp.maximum(e0, e0 * 0.2))
                        ee1 = jnp.exp(jnp.maximum(e1, e1 * 0.2))
                        evec = gi * 16 + lax.iota(jnp.int32, 16)
                        plsc.store_scatter(
                            staged, [evec, zero16 + od2], ee0)
                        plsc.store_scatter(
                            staged, [evec, zero16 + (od2 + 1)], ee1)
                        base = gi * 16
                        for j in range(16):
                            f0 = ee0[j]
                            f1 = ee1[j]
                            e = base + j
                            for k in range(od2 // 16):
                                v = rows_v[e, pl.ds(co + k * 16, 16)]
                                f = f0 if k < C2 // 16 else f1
                                staged[e, pl.ds(k * 16, 16)] = v * f
                        return ()
                    lax.fori_loop(0, CH // 16, att, ())

                    pltpu.sync_copy(staged, acc.at[dst_v], add=True)

                issue(0, bufs[0])

                def pair(i, _):
                    issue(i * 2 + 1, bufs[1])
                    consume(bufs[0])

                    @pl.when(i < npair - 1)
                    def _():
                        issue(i * 2 + 2, bufs[0])
                    consume(bufs[1])
                    return ()
                lax.fori_loop(0, npair, pair, ())

            plsc.subcore_barrier()
            ob = c * R_out + out_base[g]

            def flush(t, _):
                pltpu.sync_copy(acc.at[pl.ds(r0 + t * ZR, ZR)],
                                out_hbm.at[pl.ds(ob + r0 + t * ZR, ZR),
                                           pl.ds(0, W)])
                return ()
            lax.fori_loop(0, rslice // ZR, flush, ())
            plsc.subcore_barrier()

    return phase


_J_E = [65536, 65536, 65536, 65536, 32768]
_J_NS = [NS, NS, NM, NM, NR]
_O_E = [65536, 65536, 65536, 65536, 32768]
_O_NS = [NJ] * 5
_O_BASE = [0, NS, 2 * NS, 2 * NS + NM, 2 * NS + 2 * NM]
_O_ROWS = 2 * NS + 2 * NM + NR

_phase_j = _build_phase(_J_E, _J_NS, [0] * 5, NJ, DJ // 2)
_phase_o = _build_phase(_O_E, _O_NS, _O_BASE, _O_ROWS, DO // 2)


def _fold_a(Wmat, avec, od):
    C = od // H
    return jnp.einsum("shc,hc->sh", Wmat.reshape(-1, H, C), avec)


def _prep_tables(xs_list, xd, ps, od):
    hs_l, as_l, ad_l = [], [], []
    for xs, p in zip(xs_list, ps):
        h = xs @ p["Ws"]
        if od < 128:
            h = jnp.pad(h, ((0, 0), (0, 128 - od)))
        hs_l.append(h)
        as_l.append(xs @ _fold_a(p["Ws"], p["as"], od))
        ad_l.append(xd @ _fold_a(p["Wd"], p["ad"], od))
    hs_tab = jnp.concatenate(hs_l)
    as_tab = jnp.concatenate(
        [a[:, 2 * c:2 * c + 2] for c in (0, 1) for a in as_l])
    ad_tab = jnp.concatenate(
        [a[:, 2 * c:2 * c + 2] for c in (0, 1) for a in ad_l])
    return hs_tab, as_tab, ad_tab


def _prep_tables_o(xs, xd_list, ps, od):
    hs_l, as_l, ad_l = [], [], []
    for xd, p in zip(xd_list, ps):
        h = xs @ p["Ws"]
        if od < 128:
            h = jnp.pad(h, ((0, 0), (0, 128 - od)))
        hs_l.append(h)
        as_l.append(xs @ _fold_a(p["Ws"], p["as"], od))
        ad_l.append(xd @ _fold_a(p["Wd"], p["ad"], od))
    hs_tab = jnp.concatenate(hs_l)
    as_tab = jnp.concatenate(
        [a[:, 2 * c:2 * c + 2] for c in (0, 1) for a in as_l])
    ad_tab = jnp.concatenate(
        [a[:, 2 * c:2 * c + 2] for c in (0, 1) for a in ad_l])
    return hs_tab, as_tab, ad_tab


def _combine_j(raw, ps):
    od2 = DJ // 2
    r = raw.reshape(2, 5, NJ, 128)
    num = r[:, :, :, :od2].reshape(2, 5, NJ, 2, od2 // 2)
    den = r[:, :, :, od2:od2 + 2].reshape(2, 5, NJ, 2, 1)
    out = num / (den + 1e-16)
    out = out.reshape(2, 5, NJ, od2).transpose(1, 2, 0, 3).reshape(5, NJ, DJ)
    bsum = sum(p["b"] for p in ps)
    return out.sum(0) + bsum


def _combine_o(raw, ps):
    od2 = DO // 2
    r = raw.reshape(2, _O_ROWS, 128)
    num = r[:, :, :od2].reshape(2, _O_ROWS, 2, od2 // 2)
    den = r[:, :, od2:od2 + 2].reshape(2, _O_ROWS, 2, 1)
    out = num / (den + 1e-16)
    out = out.reshape(2, _O_ROWS, od2).transpose(1, 0, 2).reshape(_O_ROWS, DO)
    parts = []
    o = 0
    for p, n in zip(ps, [NS, NS, NM, NM, NR]):
        parts.append(out[o:o + n] + p["b"])
        o += n
    return parts


def _ln(x, g, b):
    mu = x.mean(-1, keepdims=True)
    v = ((x - mu) ** 2).mean(-1, keepdims=True)
    return (x - mu) / jnp.sqrt(v + 1e-5) * g + b


def _mlp_body(feat_ref, w1_ref, b1_ref, w2_ref, b2_ref, w3_ref, b3_ref, o_ref):
    h = jnp.maximum(feat_ref[...] @ w1_ref[...] + b1_ref[...], 0.0)
    h = jnp.maximum(h @ w2_ref[...] + b2_ref[...], 0.0)
    o_ref[...] = h @ w3_ref[...] + b3_ref[...]


def _q_mlp(feat, P):
    w3 = jnp.pad(P["q3_W"], ((0, 0), (0, 127)))
    b3 = jnp.pad(P["q3_b"], (0, 127))
    out = pl.pallas_call(
        _mlp_body,
        out_shape=jax.ShapeDtypeStruct((A, 128), jnp.float32),
    )(feat, P["q1_W"], P["q1_b"], P["q2_W"], P["q2_b"], w3, b3)
    return out[:, 0]


_J_RELS = ("cl", "ld", "we", "ex", "hd")
_O_RELS = ("cbl", "li", "nd", "eb", "hb")


def kernel(x_job, x_station, x_machine, x_robot, alpha, actions, params, edges):
    P = params
    relu = jax.nn.relu
    hj = relu(x_job @ P["lj_W"] + P["lj_b"])
    hs = relu(x_station @ P["ls_W"] + P["ls_b"])
    hm = relu(x_machine @ P["lm_W"] + P["lm_b"])
    hr = relu(x_robot @ P["lr_W"] + P["lr_b"])

    jt = [0, NS, NS, 0, 0]
    j_tb = [0, NS, 2 * NS, 2 * NS + NM, 2 * NS + 2 * NM]
    src_j = jnp.concatenate(
        [edges[r][0].astype(jnp.int32) + j_tb[i] for i, r in enumerate(_J_RELS)])
    dst_j = jnp.concatenate(
        [edges[r][1].astype(jnp.int32) for r in _J_RELS])
    o_tb = [NJ * i for i in range(5)]
    src_o = jnp.concatenate(
        [edges[r][0].astype(jnp.int32) + o_tb[i] for i, r in enumerate(_O_RELS)])
    dst_o = jnp.concatenate(
        [edges[r][1].astype(jnp.int32) + _O_BASE[i] for i, r in enumerate(_O_RELS)])

    for jl, ol in (("j1", "o1"), ("j2", "o2")):
        pj = P[jl]
        ps_j = [pj[r] for r in _J_RELS]
        hs_tab, as_tab, ad_tab = _prep_tables([hs, hs, hm, hm, hr], hj, ps_j, DJ)
        raw = _phase_j(src_j, dst_j, hs_tab,
                       as_tab.reshape(-1), ad_tab.reshape(-1))
        msg = _combine_j(raw, ps_j)
        hj = _ln(relu(msg + hj), pj["ln_g"], pj["ln_b"])

        po = P[ol]
        ps_o = [po[r] for r in _O_RELS]
        hs_tab, as_tab, ad_tab = _prep_tables_o(hj, [hs, hs, hm, hm, hr], ps_o, DO)
        raw = _phase_o(src_o, dst_o, hs_tab,
                       as_tab.reshape(-1), ad_tab.reshape(-1))
        mss, mli, mnd, meb, mhb = _combine_o(raw, ps_o)
        ms = mss + mli
        mm = mnd + meb
        mr = mhb
        hs = _ln(relu(ms + hs), po["ln_gs"], po["ln_bs"])
        hm = _ln(relu(mm + hm), po["ln_gm"], po["ln_bm"])
        hr = _ln(relu(mr + hr), po["ln_gr"], po["ln_br"])

    h_nodes = jnp.concatenate(
        [hs.reshape(B, 3 * DO), hm.reshape(B, 2 * DO), hr.reshape(B, DO)], axis=1)
    gate = (hj @ P["gate_W"] + P["gate_b"])[:, 0].reshape(B, J)
    ge = jnp.exp(gate)
    w = ge / (ge.sum(-1, keepdims=True) + 1e-16)
    mean_jobs = (hj.reshape(B, J, DJ) * w[:, :, None]).sum(1)
    h_global = relu(jnp.concatenate([h_nodes, mean_jobs], axis=1) @ P["gl_W"] + P["gl_b"])
    job_ids = actions[:, 0]
    graph_ids = job_ids // J
    gji = job_ids + graph_ids * J
    emb = hj[gji]
    hg = h_global[graph_ids]
    aA = jnp.broadcast_to(alpha.reshape(1, 1).astype(jnp.float32), (A, 1))
    feat = jnp.concatenate([emb, hg, actions[:, 1:2].astype(jnp.float32),
                            actions[:, 2:3].astype(jnp.float32), aA], axis=1)
    return _q_mlp(feat, P)


# speedup vs baseline: 58.1753x; 1.4053x over previous
import functools

import jax
import jax.numpy as jnp
from jax import lax
from jax.experimental import pallas as pl
from jax.experimental.pallas import tpu as pltpu
from jax.experimental.pallas import tpu_sc as plsc

B = 1024; J = 16; NJ = B * J; NS = 3 * B; NM = 2 * B; NR = B
DJ = 128; DO = 64; H = 4; GD = 128; A = 4096

CH = 64
ZR = 64


def _build_phase(E_list, ns_list, acc_base, rows_g, od2):
    W = od2 + 16
    nrel = len(E_list)
    tab_base = [0]
    for ns in ns_list[:-1]:
        tab_base.append(tab_base[-1] + ns)
    T = tab_base[-1] + ns_list[-1]
    jstyle = all(b == 0 for b in acc_base)
    if jstyle:
        groups = [[r] for r in range(nrel)]
        ad_base = [rows_g * r for r in range(nrel)]
        ADR = rows_g * nrel
        out_base = [rows_g * r for r in range(nrel)]
        R_out = rows_g * nrel
    else:
        groups = [list(range(nrel))]
        ad_base = [0] * nrel
        ADR = rows_g
        out_base = [0]
        R_out = rows_g
    e_base = [0]
    for E in E_list[:-1]:
        e_base.append(e_base[-1] + E)
    MAXNS = max(ns_list)
    C2 = od2 // 2

    mesh = plsc.VectorSubcoreMesh(core_axis_name="c", subcore_axis_name="s")

    @functools.partial(
        pl.kernel, mesh=mesh,
        compiler_params=pltpu.CompilerParams(
            needs_layout_passes=False, use_tc_tiling_on_sc=False),
        out_type=jax.ShapeDtypeStruct((2 * R_out, 128), jnp.float32),
        scratch_types=[
            pltpu.VMEM((CH, 128), jnp.float32),
            pltpu.VMEM((CH, 128), jnp.float32),
            pltpu.VMEM((CH, W), jnp.float32),
            pltpu.VMEM((CH, W), jnp.float32),
            pltpu.VMEM((4096,), jnp.int32),
            pltpu.VMEM((4096,), jnp.int32),
            pltpu.VMEM((CH,), jnp.int32),
            pltpu.VMEM((CH,), jnp.int32),
            pltpu.VMEM((CH,), jnp.int32),
            pltpu.VMEM((CH,), jnp.int32),
            pltpu.VMEM((CH,), jnp.int32),
            pltpu.VMEM((CH,), jnp.int32),
            pltpu.VMEM((CH,), jnp.int32),
            pltpu.VMEM((CH,), jnp.int32),
            pltpu.VMEM((CH,), jnp.int32),
            pltpu.VMEM((CH,), jnp.int32),
            pltpu.VMEM((CH,), jnp.float32),
            pltpu.VMEM((CH,), jnp.float32),
            pltpu.VMEM((CH,), jnp.float32),
            pltpu.VMEM((CH,), jnp.float32),
            pltpu.VMEM((ZR, W), jnp.float32),
            pltpu.VMEM((MAXNS * 2,), jnp.float32),
            pltpu.VMEM_SHARED((rows_g * 2,), jnp.float32),
            pltpu.VMEM_SHARED((rows_g, W), jnp.float32),
            pltpu.SemaphoreType.DMA,
            pltpu.SemaphoreType.DMA,
            pltpu.SemaphoreType.DMA,
            pltpu.SemaphoreType.DMA,
            pltpu.SemaphoreType.DMA,
            pltpu.SemaphoreType.DMA,
        ])
    def phase(src_hbm, dst_hbm, hs_hbm, as_hbm, ad_hbm, out_hbm,
              rows_v0, rows_v1, staged0, staged1, src_t, dst_t,
              src_v0, src_v1, dst_v0, dst_v1, dsc0, dsc1,
              i0b0, i0b1, i1b0, i1b1, ad0v0, ad0v1, ad1v0, ad1v1,
              zbuf, asb, adsh, acc,
              semr0, semr1, sema0, sema1, semw0, semw1):
        c = lax.axis_index("c")
        s = lax.axis_index("s")
        cT = c * T
        co = c * od2

        def zrow(i, _):
            for k in range(W // 16):
                zbuf[i, pl.ds(k * 16, 16)] = jnp.zeros((16,), jnp.float32)
            return ()
        lax.fori_loop(0, ZR, zrow, ())

        def zpad(i, _):
            staged0[i, pl.ds(od2, 16)] = jnp.zeros((16,), jnp.float32)
            staged1[i, pl.ds(od2, 16)] = jnp.zeros((16,), jnp.float32)
            return ()
        lax.fori_loop(0, CH, zpad, ())

        for g, rels in enumerate(groups):
            rg = rows_g
            rslice = rg // 16
            r0 = s * rslice

            def zacc(t, _):
                pltpu.sync_copy(zbuf, acc.at[pl.ds(r0 + t * ZR, ZR)])
                return ()
            lax.fori_loop(0, rslice // ZR, zacc, ())
            adr0 = s * (rg * 2 // 16)
            pltpu.sync_copy(
                ad_hbm.at[pl.ds((c * ADR + ad_base[g]) * 2 + adr0,
                                rg * 2 // 16)],
                adsh.at[pl.ds(adr0, rg * 2 // 16)])
            plsc.subcore_barrier()

            for r in rels:
                E = E_list[r]; ns = ns_list[r]; tb = tab_base[r]
                ept = E // 16
                eb = e_base[r] + s * ept
                nch = ept // CH
                npair = nch // 2
                pltpu.sync_copy(as_hbm.at[pl.ds((cT + tb) * 2, ns * 2)],
                                asb.at[pl.ds(0, ns * 2)])
                pltpu.sync_copy(src_hbm.at[pl.ds(eb, ept)],
                                src_t.at[pl.ds(0, ept)])
                pltpu.sync_copy(dst_hbm.at[pl.ds(eb, ept)],
                                dst_t.at[pl.ds(0, ept)])
                bufs = (
                    (rows_v0, staged0, src_v0, dst_v0, dsc0,
                     i0b0, i1b0, ad0v0, ad1v0, semr0, sema0, semw0),
                    (rows_v1, staged1, src_v1, dst_v1, dsc1,
                     i0b1, i1b1, ad0v1, ad1v1, semr1, sema1, semw1),
                )

                def issue(ci, bf):
                    (rows_v, staged, src_v, dst_v, dsc, i0b, i1b,
                     ad0v, ad1v, smr, sma, smw) = bf

                    def mkidx(j, _):
                        o = ci * CH + j * 16
                        s16 = src_t[pl.ds(o, 16)]
                        d16 = dst_t[pl.ds(o, 16)]
                        src_v[pl.ds(j * 16, 16)] = s16
                        dst_v[pl.ds(j * 16, 16)] = d16
                        i0b[pl.ds(j * 16, 16)] = d16 * 2
                        i1b[pl.ds(j * 16, 16)] = d16 * 2 + 1
                        return ()
                    lax.fori_loop(0, CH // 16, mkidx, ())
                    pltpu.async_copy(hs_hbm.at[src_v], rows_v, smr)
                    pltpu.async_copy(adsh.at[i0b], ad0v, sma)
                    pltpu.async_copy(adsh.at[i1b], ad1v, sma)

                def consume(ci, bf):
                    (rows_v, staged, src_v, dst_v, dsc, i0b, i1b,
                     ad0v, ad1v, smr, sma, smw) = bf

                    @pl.when(ci >= 2)
                    def _():
                        pltpu.make_async_copy(
                            staged, acc.at[dsc], smw).wait()
                    pltpu.make_async_copy(hs_hbm.at[src_v], rows_v, smr).wait()
                    pltpu.make_async_copy(adsh.at[i0b], ad0v, sma).wait()
                    pltpu.make_async_copy(adsh.at[i1b], ad1v, sma).wait()

                    zero16 = jnp.zeros((16,), jnp.int32)

                    def att(gi, _):
                        s16 = src_v[pl.ds(gi * 16, 16)] - tb
                        as0 = plsc.load_gather(asb, [s16 * 2])
                        as1 = plsc.load_gather(asb, [s16 * 2 + 1])
                        ad0 = ad0v[pl.ds(gi * 16, 16)]
                        ad1 = ad1v[pl.ds(gi * 16, 16)]
                        e0 = as0 + ad0
                        e1 = as1 + ad1
                        ee0 = jnp.exp(jnp.maximum(e0, e0 * 0.2))
                        ee1 = jnp.exp(jnp.maximum(e1, e1 * 0.2))
                        evec = gi * 16 + lax.iota(jnp.int32, 16)
                        plsc.store_scatter(
                            staged, [evec, zero16 + od2], ee0)
                        plsc.store_scatter(
                            staged, [evec, zero16 + (od2 + 1)], ee1)
                        dsc[pl.ds(gi * 16, 16)] = dst_v[pl.ds(gi * 16, 16)]
                        base = gi * 16
                        for j in range(16):
                            f0 = ee0[j]
                            f1 = ee1[j]
                            e = base + j
                            for k in range(od2 // 16):
                                v = rows_v[e, pl.ds(co + k * 16, 16)]
                                f = f0 if k < C2 // 16 else f1
                                staged[e, pl.ds(k * 16, 16)] = v * f
                        return ()
                    lax.fori_loop(0, CH // 16, att, ())

                    pltpu.async_copy(staged, acc.at[dsc], smw, add=True)

                issue(0, bufs[0])
                issue(1, bufs[1])

                def pair(i, _):
                    consume(i * 2, bufs[0])

                    @pl.when(i < npair - 1)
                    def _():
                        issue(i * 2 + 2, bufs[0])
                    consume(i * 2 + 1, bufs[1])

                    @pl.when(i < npair - 1)
                    def _():
                        issue(i * 2 + 3, bufs[1])
                    return ()
                lax.fori_loop(0, npair, pair, ())
                pltpu.make_async_copy(staged0, acc.at[dsc0], semw0).wait()
                pltpu.make_async_copy(staged1, acc.at[dsc1], semw1).wait()

            plsc.subcore_barrier()
            ob = c * R_out + out_base[g]

            def flush(t, _):
                pltpu.sync_copy(acc.at[pl.ds(r0 + t * ZR, ZR)],
                                out_hbm.at[pl.ds(ob + r0 + t * ZR, ZR),
                                           pl.ds(0, W)])
                return ()
            lax.fori_loop(0, rslice // ZR, flush, ())
            plsc.subcore_barrier()

    return phase


_J_E = [65536, 65536, 65536, 65536, 32768]
_J_NS = [NS, NS, NM, NM, NR]
_O_E = [65536, 65536, 65536, 65536, 32768]
_O_NS = [NJ] * 5
_O_BASE = [0, NS, 2 * NS, 2 * NS + NM, 2 * NS + 2 * NM]
_O_ROWS = 2 * NS + 2 * NM + NR

_phase_j = _build_phase(_J_E, _J_NS, [0] * 5, NJ, DJ // 2)
_phase_o = _build_phase(_O_E, _O_NS, _O_BASE, _O_ROWS, DO // 2)


def _fold_a(Wmat, avec, od):
    C = od // H
    return jnp.einsum("shc,hc->sh", Wmat.reshape(-1, H, C), avec)


def _prep_tables(xs_list, xd, ps, od):
    hs_l, as_l, ad_l = [], [], []
    for xs, p in zip(xs_list, ps):
        h = xs @ p["Ws"]
        if od < 128:
            h = jnp.pad(h, ((0, 0), (0, 128 - od)))
        hs_l.append(h)
        as_l.append(xs @ _fold_a(p["Ws"], p["as"], od))
        ad_l.append(xd @ _fold_a(p["Wd"], p["ad"], od))
    hs_tab = jnp.concatenate(hs_l)
    as_tab = jnp.concatenate(
        [a[:, 2 * c:2 * c + 2] for c in (0, 1) for a in as_l])
    ad_tab = jnp.concatenate(
        [a[:, 2 * c:2 * c + 2] for c in (0, 1) for a in ad_l])
    return hs_tab, as_tab, ad_tab


def _prep_tables_o(xs, xd_list, ps, od):
    hs_l, as_l, ad_l = [], [], []
    for xd, p in zip(xd_list, ps):
        h = xs @ p["Ws"]
        if od < 128:
            h = jnp.pad(h, ((0, 0), (0, 128 - od)))
        hs_l.append(h)
        as_l.append(xs @ _fold_a(p["Ws"], p["as"], od))
        ad_l.append(xd @ _fold_a(p["Wd"], p["ad"], od))
    hs_tab = jnp.concatenate(hs_l)
    as_tab = jnp.concatenate(
        [a[:, 2 * c:2 * c + 2] for c in (0, 1) for a in as_l])
    ad_tab = jnp.concatenate(
        [a[:, 2 * c:2 * c + 2] for c in (0, 1) for a in ad_l])
    return hs_tab, as_tab, ad_tab


def _combine_j(raw, ps):
    od2 = DJ // 2
    r = raw.reshape(2, 5, NJ, 128)
    num = r[:, :, :, :od2].reshape(2, 5, NJ, 2, od2 // 2)
    den = r[:, :, :, od2:od2 + 2].reshape(2, 5, NJ, 2, 1)
    out = num / (den + 1e-16)
    out = out.reshape(2, 5, NJ, od2).transpose(1, 2, 0, 3).reshape(5, NJ, DJ)
    bsum = sum(p["b"] for p in ps)
    return out.sum(0) + bsum


def _combine_o(raw, ps):
    od2 = DO // 2
    r = raw.reshape(2, _O_ROWS, 128)
    num = r[:, :, :od2].reshape(2, _O_ROWS, 2, od2 // 2)
    den = r[:, :, od2:od2 + 2].reshape(2, _O_ROWS, 2, 1)
    out = num / (den + 1e-16)
    out = out.reshape(2, _O_ROWS, od2).transpose(1, 0, 2).reshape(_O_ROWS, DO)
    parts = []
    o = 0
    for p, n in zip(ps, [NS, NS, NM, NM, NR]):
        parts.append(out[o:o + n] + p["b"])
        o += n
    return parts


def _ln(x, g, b):
    mu = x.mean(-1, keepdims=True)
    v = ((x - mu) ** 2).mean(-1, keepdims=True)
    return (x - mu) / jnp.sqrt(v + 1e-5) * g + b


def _mlp_body(feat_ref, w1_ref, b1_ref, w2_ref, b2_ref, w3_ref, b3_ref, o_ref):
    h = jnp.maximum(feat_ref[...] @ w1_ref[...] + b1_ref[...], 0.0)
    h = jnp.maximum(h @ w2_ref[...] + b2_ref[...], 0.0)
    o_ref[...] = h @ w3_ref[...] + b3_ref[...]


def _q_mlp(feat, P):
    w3 = jnp.pad(P["q3_W"], ((0, 0), (0, 127)))
    b3 = jnp.pad(P["q3_b"], (0, 127))
    out = pl.pallas_call(
        _mlp_body,
        out_shape=jax.ShapeDtypeStruct((A, 128), jnp.float32),
    )(feat, P["q1_W"], P["q1_b"], P["q2_W"], P["q2_b"], w3, b3)
    return out[:, 0]


_J_RELS = ("cl", "ld", "we", "ex", "hd")
_O_RELS = ("cbl", "li", "nd", "eb", "hb")


def kernel(x_job, x_station, x_machine, x_robot, alpha, actions, params, edges):
    P = params
    relu = jax.nn.relu
    hj = relu(x_job @ P["lj_W"] + P["lj_b"])
    hs = relu(x_station @ P["ls_W"] + P["ls_b"])
    hm = relu(x_machine @ P["lm_W"] + P["lm_b"])
    hr = relu(x_robot @ P["lr_W"] + P["lr_b"])

    jt = [0, NS, NS, 0, 0]
    j_tb = [0, NS, 2 * NS, 2 * NS + NM, 2 * NS + 2 * NM]
    src_j = jnp.concatenate(
        [edges[r][0].astype(jnp.int32) + j_tb[i] for i, r in enumerate(_J_RELS)])
    dst_j = jnp.concatenate(
        [edges[r][1].astype(jnp.int32) for r in _J_RELS])
    o_tb = [NJ * i for i in range(5)]
    src_o = jnp.concatenate(
        [edges[r][0].astype(jnp.int32) + o_tb[i] for i, r in enumerate(_O_RELS)])
    dst_o = jnp.concatenate(
        [edges[r][1].astype(jnp.int32) + _O_BASE[i] for i, r in enumerate(_O_RELS)])

    for jl, ol in (("j1", "o1"), ("j2", "o2")):
        pj = P[jl]
        ps_j = [pj[r] for r in _J_RELS]
        hs_tab, as_tab, ad_tab = _prep_tables([hs, hs, hm, hm, hr], hj, ps_j, DJ)
        raw = _phase_j(src_j, dst_j, hs_tab,
                       as_tab.reshape(-1), ad_tab.reshape(-1))
        msg = _combine_j(raw, ps_j)
        hj = _ln(relu(msg + hj), pj["ln_g"], pj["ln_b"])

        po = P[ol]
        ps_o = [po[r] for r in _O_RELS]
        hs_tab, as_tab, ad_tab = _prep_tables_o(hj, [hs, hs, hm, hm, hr], ps_o, DO)
        raw = _phase_o(src_o, dst_o, hs_tab,
                       as_tab.reshape(-1), ad_tab.reshape(-1))
        mss, mli, mnd, meb, mhb = _combine_o(raw, ps_o)
        ms = mss + mli
        mm = mnd + meb
        mr = mhb
        hs = _ln(relu(ms + hs), po["ln_gs"], po["ln_bs"])
        hm = _ln(relu(mm + hm), po["ln_gm"], po["ln_bm"])
        hr = _ln(relu(mr + hr), po["ln_gr"], po["ln_br"])

    h_nodes = jnp.concatenate(
        [hs.reshape(B, 3 * DO), hm.reshape(B, 2 * DO), hr.reshape(B, DO)], axis=1)
    gate = (hj @ P["gate_W"] + P["gate_b"])[:, 0].reshape(B, J)
    ge = jnp.exp(gate)
    w = ge / (ge.sum(-1, keepdims=True) + 1e-16)
    mean_jobs = (hj.reshape(B, J, DJ) * w[:, :, None]).sum(1)
    h_global = relu(jnp.concatenate([h_nodes, mean_jobs], axis=1) @ P["gl_W"] + P["gl_b"])
    job_ids = actions[:, 0]
    graph_ids = job_ids // J
    gji = job_ids + graph_ids * J
    emb = hj[gji]
    hg = h_global[graph_ids]
    aA = jnp.broadcast_to(alpha.reshape(1, 1).astype(jnp.float32), (A, 1))
    feat = jnp.concatenate([emb, hg, actions[:, 1:2].astype(jnp.float32),
                            actions[:, 2:3].astype(jnp.float32), aA], axis=1)
    return _q_mlp(feat, P)
